# Initial kernel scaffold; baseline (speedup 1.0000x reference)
import jax
import jax.numpy as jnp
from jax.experimental import pallas as pl


def kernel(cls_logits, bbox_pred, priors):
    raise NotImplementedError("write your pallas kernel here")


# speedup vs baseline: 1.0001x; 1.0001x over previous
import jax
import jax.numpy as jnp
from jax.experimental import pallas as pl

CENTER_VAR = 0.1
SIZE_VAR = 0.2
SCORE_THR = 0.01
IOU_THR = 0.45
K = 200
MAX_DET = 100


def kernel(cls_logits, bbox_pred, priors):
    cxcy = bbox_pred[..., :2] * CENTER_VAR * priors[..., 2:] + priors[..., :2]
    wh = jnp.exp(bbox_pred[..., 2:] * SIZE_VAR) * priors[..., 2:]
    boxes_c = jnp.concatenate([cxcy, wh], axis=-1)
    scores = jax.nn.softmax(cls_logits, axis=2)
    boxes = jnp.concatenate([boxes_c[..., :2] - boxes_c[..., 2:] / 2.0,
                             boxes_c[..., :2] + boxes_c[..., 2:] / 2.0], axis=-1)
    cls_scores = jnp.transpose(scores[:, :, 1:], (0, 2, 1))
    topv, topi = jax.lax.top_k(cls_scores, K)
    cand_boxes = jnp.take_along_axis(boxes[:, None, :, :], topi[..., None], axis=2)
    area = (cand_boxes[..., 2] - cand_boxes[..., 0]) * (cand_boxes[..., 3] - cand_boxes[..., 1])
    lt = jnp.maximum(cand_boxes[..., :, None, :2], cand_boxes[..., None, :, :2])
    rb = jnp.minimum(cand_boxes[..., :, None, 2:], cand_boxes[..., None, :, 2:])
    whp = jnp.clip(rb - lt, 0.0)
    inter = whp[..., 0] * whp[..., 1]
    union = area[..., :, None] + area[..., None, :] - inter
    iou = inter / jnp.maximum(union, 1e-8)
    higher = topv[..., None, :] > topv[..., :, None]
    suppressed = jnp.any((iou > IOU_THR) & higher, axis=-1)
    keep = jnp.logical_and(jnp.logical_not(suppressed), topv > SCORE_THR)
    kept_scores = topv * keep.astype(topv.dtype)
    flat_scores = kept_scores.reshape(kept_scores.shape[0], -1)
    flat_boxes = cand_boxes.reshape(cand_boxes.shape[0], -1, 4)
    fs, fi = jax.lax.top_k(flat_scores, MAX_DET)
    fb = jnp.take_along_axis(flat_boxes, fi[..., None], axis=1)
    return jnp.concatenate([fb, fs[..., None]], axis=-1)


# speedup vs baseline: 4.7262x; 4.7258x over previous
import functools

import jax
import jax.numpy as jnp
from jax import lax
from jax.experimental import pallas as pl
from jax.experimental.pallas import tpu as pltpu
from jax.experimental.pallas import tpu_sc as plsc

CENTER_VAR = 0.1
SIZE_VAR = 0.2
SCORE_THR = 0.01
IOU_THR = 0.45
K = 200
MAX_DET = 100
B, N, C = 8, 20000, 81
NCLS = C - 1
NROWS = B * NCLS
L = 16
NWORKERS = 32


def _iota16():
    return lax.broadcasted_iota(jnp.int32, (L,), 0)


def _splat_to_scalar(v):
    return v[0]


def _radix_select_topk(row_val, row_idx, hist, scnt, out_val, out_idx, m_total, k_sel):
    lanes = _iota16()
    ones = jnp.ones((L,), jnp.int32)
    lane_base = lanes * 256

    m_cur = m_total
    k_rem = k_sel
    out_off = jnp.int32(0)

    for level, shift in enumerate((24, 16, 8, 0)):
        def zero_body(j, _):
            hist[pl.ds(j * L, L)] = jnp.zeros((L,), jnp.int32)
            return 0
        lax.fori_loop(0, 256, zero_body, 0)

        def hist_body(j, _, shift=shift):
            v = row_val[pl.ds(j * L, L)]
            key = lax.bitcast_convert_type(v, jnp.int32)
            digit = lax.shift_right_logical(key, shift) & 255
            valid = (j * L + lanes) < m_cur
            plsc.addupdate_scatter(hist, [digit + lane_base], ones, mask=valid)
            return 0
        n_chunks = (m_cur + (L - 1)) // L
        lax.fori_loop(0, n_chunks, hist_body, 0)

        def suffix_body(i, carry_and_count):
            carry, count_ge = carry_and_count
            d = 15 - i
            tot = hist[pl.ds(d * L, L)]
            for l in range(1, L):
                tot = tot + hist[pl.ds(l * 256 + d * L, L)]
            sfx = lax.rev(plsc.cumsum(lax.rev(tot, (0,))), (0,)) + carry
            scnt[pl.ds(d * L, L)] = sfx
            carry = jnp.broadcast_to(sfx[0], (L,))
            ge = sfx >= jnp.broadcast_to(k_rem, (L,))
            count_ge = count_ge + _splat_to_scalar(plsc.all_reduce_population_count(ge))
            return carry, count_ge
        _, count_ge = lax.fori_loop(
            0, 16, suffix_body, (jnp.zeros((L,), jnp.int32), jnp.int32(0)))
        t = count_ge - 1
        nxt = scnt[pl.ds(t + 1, L)]
        cnt_gt = jnp.max(nxt)
        m_next_sel = k_rem - cnt_gt

        t_splat = jnp.broadcast_to(t, (L,))

        def compact_body(j, offs, shift=shift, level=level):
            ooff, eoff = offs
            v = row_val[pl.ds(j * L, L)]
            key = lax.bitcast_convert_type(v, jnp.int32)
            digit = lax.shift_right_logical(key, shift) & 255
            if level == 0:
                idx = j * L + lanes
            else:
                idx = row_idx[pl.ds(j * L, L)]
            valid = (j * L + lanes) < m_cur
            sel = (digit > t_splat) & valid
            eq = (digit == t_splat) & valid
            plsc.store_compressed(out_val.at[pl.ds(ooff, L)], v, mask=sel)
            plsc.store_compressed(out_idx.at[pl.ds(ooff, L)], idx, mask=sel)
            plsc.store_compressed(row_val.at[pl.ds(eoff, L)], v, mask=eq)
            plsc.store_compressed(row_idx.at[pl.ds(eoff, L)], idx, mask=eq)
            ooff = ooff + _splat_to_scalar(plsc.all_reduce_population_count(sel))
            eoff = eoff + _splat_to_scalar(plsc.all_reduce_population_count(eq))
            return ooff, eoff
        out_off, e_off = lax.fori_loop(
            0, n_chunks, compact_body, (out_off, jnp.int32(0)))
        m_cur = e_off
        k_rem = m_next_sel

    def tie_body(j, ooff):
        v = row_val[pl.ds(j * L, L)]
        idx = row_idx[pl.ds(j * L, L)]
        take = (j * L + lanes) < k_rem
        plsc.store_compressed(out_val.at[pl.ds(ooff, L)], v, mask=take)
        plsc.store_compressed(out_idx.at[pl.ds(ooff, L)], idx, mask=take)
        return ooff + _splat_to_scalar(plsc.all_reduce_population_count(take))
    lax.fori_loop(0, (k_rem + (L - 1)) // L, tie_body, out_off)


def _k2_body(scores_hbm, boxes_hbm, cand_val_hbm, cand_boxes_hbm,
             row_val, row_idx, hist, scnt, out_val, out_idx,
             idx_a, idx_b, boxbuf_a, boxbuf_b, sem):
    wid = lax.axis_index("s") * 2 + lax.axis_index("c")
    rows_per = NROWS // NWORKERS
    b = (wid * rows_per) // NCLS

    def z_body(j, _):
        scnt[pl.ds(256 + j * L, L)] = jnp.zeros((L,), jnp.int32)
        out_idx[pl.ds(K + j * L, L)] = jnp.zeros((L,), jnp.int32)
        return 0
    lax.fori_loop(0, 1, z_body, 0)

    def row_body(i, _):
        r = wid * rows_per + i
        cls = r % NCLS + 1
        pltpu.sync_copy(scores_hbm.at[b, cls], row_val)
        _radix_select_topk(row_val, row_idx, hist, scnt, out_val, out_idx,
                           jnp.int32(N), jnp.int32(K))
        def i_body(j, _):
            idx_a[pl.ds(j * L, L)] = out_idx[pl.ds(j * L, L)]
            idx_b[pl.ds(j * L, L)] = out_idx[pl.ds(112 + j * L, L)]
            return 0
        lax.fori_loop(0, 112 // L, i_body, 0)
        ca = pltpu.make_async_copy(boxes_hbm.at[b].at[idx_a], boxbuf_a, sem)
        cb_ = pltpu.make_async_copy(boxes_hbm.at[b].at[idx_b], boxbuf_b, sem)
        ca.start()
        cb_.start()
        ca.wait()
        cb_.wait()
        pltpu.sync_copy(out_val.at[pl.ds(0, K)], cand_val_hbm.at[r])
        pltpu.sync_copy(boxbuf_a.at[:, pl.ds(0, 4)],
                        cand_boxes_hbm.at[r, pl.ds(0, 112)])
        pltpu.sync_copy(boxbuf_b.at[pl.ds(0, K - 112), pl.ds(0, 4)],
                        cand_boxes_hbm.at[r, pl.ds(112, K - 112)])
        return 0
    lax.fori_loop(0, rows_per, row_body, 0)


@jax.jit
def _k2_select(scores_t, boxes):
    mesh = plsc.VectorSubcoreMesh(core_axis_name="c", subcore_axis_name="s",
                                  num_cores=2, num_subcores=16)
    f = pl.kernel(
        _k2_body,
        out_type=[jax.ShapeDtypeStruct((NROWS, K), jnp.float32),
                  jax.ShapeDtypeStruct((NROWS, K, 4), jnp.float32)],
        mesh=mesh,
        scratch_types=[
            pltpu.VMEM((N,), jnp.float32),
            pltpu.VMEM((N,), jnp.int32),
            pltpu.VMEM((16 * 256,), jnp.int32),
            pltpu.VMEM((272,), jnp.int32),
            pltpu.VMEM((224,), jnp.float32),
            pltpu.VMEM((224,), jnp.int32),
            pltpu.VMEM((112,), jnp.int32),
            pltpu.VMEM((112,), jnp.int32),
            pltpu.VMEM((112, 16), jnp.float32),
            pltpu.VMEM((112, 16), jnp.float32),
            pltpu.SemaphoreType.DMA,
        ],
        compiler_params=pltpu.CompilerParams(needs_layout_passes=False,
                                             use_tc_tiling_on_sc=False),
    )
    return f(scores_t, boxes)


def kernel(cls_logits, bbox_pred, priors):
    cxcy = bbox_pred[..., :2] * CENTER_VAR * priors[..., 2:] + priors[..., :2]
    wh = jnp.exp(bbox_pred[..., 2:] * SIZE_VAR) * priors[..., 2:]
    boxes = jnp.concatenate([cxcy - wh / 2.0, cxcy + wh / 2.0], axis=-1)
    boxes16 = jnp.pad(boxes, ((0, 0), (0, 0), (0, 12)))
    scores_t = jnp.transpose(jax.nn.softmax(cls_logits, axis=2), (0, 2, 1))

    topv, cand_boxes = _k2_select(scores_t, boxes16)
    topv = topv.reshape(B, NCLS, K)
    cand_boxes = cand_boxes.reshape(B, NCLS, K, 4)

    area = (cand_boxes[..., 2] - cand_boxes[..., 0]) * (cand_boxes[..., 3] - cand_boxes[..., 1])
    lt = jnp.maximum(cand_boxes[..., :, None, :2], cand_boxes[..., None, :, :2])
    rb = jnp.minimum(cand_boxes[..., :, None, 2:], cand_boxes[..., None, :, 2:])
    whp = jnp.clip(rb - lt, 0.0)
    inter = whp[..., 0] * whp[..., 1]
    union = area[..., :, None] + area[..., None, :] - inter
    iou = inter / jnp.maximum(union, 1e-8)
    higher = topv[..., None, :] > topv[..., :, None]
    suppressed = jnp.any((iou > IOU_THR) & higher, axis=-1)
    keep = jnp.logical_and(jnp.logical_not(suppressed), topv > SCORE_THR)
    kept_scores = topv * keep.astype(topv.dtype)

    flat_scores = kept_scores.reshape(B, -1)
    flat_boxes = cand_boxes.reshape(B, -1, 4)
    fs, fi = jax.lax.top_k(flat_scores, MAX_DET)
    fb = jnp.take_along_axis(flat_boxes, fi[..., None], axis=1)
    return jnp.concatenate([fb, fs[..., None]], axis=-1)
